# Initial kernel scaffold; baseline (speedup 1.0000x reference)
import jax
import jax.numpy as jnp
from jax.experimental import pallas as pl


def kernel(x, edge_index, W1, b1, W2, b2):
    raise NotImplementedError("write your pallas kernel here")


# speedup vs baseline: 101.2728x; 101.2728x over previous
import functools

import jax
import jax.numpy as jnp
from jax import lax
from jax.experimental import pallas as pl
from jax.experimental.pallas import tpu as pltpu
from jax.experimental.pallas import tpu_sc as plsc

NC = 2
NS = 16
NT = NC * NS
CHUNK = 2048


def _cdiv(a, b):
    return (a + b - 1) // b


def _make_deg_kernel(EP, NP, SL, EPT):
    mesh = plsc.VectorSubcoreMesh(
        core_axis_name="c", subcore_axis_name="s", num_cores=NC,
        num_subcores=NS)

    @functools.partial(
        pl.kernel,
        out_type=jax.ShapeDtypeStruct((NC, NP), jnp.float32),
        mesh=mesh,
        scratch_types=[
            pltpu.VMEM((CHUNK,), jnp.int32),
            pltpu.VMEM((CHUNK,), jnp.float32),
            pltpu.VMEM_SHARED((NP,), jnp.float32),
        ],
    )
    def deg_kernel(dst_hbm, ones_hbm, zeros_hbm, out_hbm, idx_v, ones_v,
                   acc_sh):
        cid = lax.axis_index("c")
        sid = lax.axis_index("s")
        pltpu.sync_copy(ones_hbm, ones_v)
        pltpu.sync_copy(zeros_hbm.at[pl.ds(sid * SL, SL)],
                        acc_sh.at[pl.ds(sid * SL, SL)])
        plsc.subcore_barrier()
        base = (cid * NS + sid) * EPT

        def body(k, carry):
            off = base + k * CHUNK
            pltpu.sync_copy(dst_hbm.at[pl.ds(off, CHUNK)], idx_v)
            pltpu.sync_copy(ones_v, acc_sh.at[idx_v], add=True)
            return carry

        lax.fori_loop(0, EPT // CHUNK, body, 0)
        plsc.subcore_barrier()
        pltpu.sync_copy(acc_sh.at[pl.ds(sid * SL, SL)],
                        out_hbm.at[cid, pl.ds(sid * SL, SL)])

    return deg_kernel


def _make_edge_kernel(EP, NP, SL, EPT):
    mesh = plsc.VectorSubcoreMesh(
        core_axis_name="c", subcore_axis_name="s", num_cores=NC,
        num_subcores=NS)

    @functools.partial(
        pl.kernel,
        out_type=(jax.ShapeDtypeStruct((NC, NP), jnp.float32),
                  jax.ShapeDtypeStruct((NC, NP), jnp.float32)),
        mesh=mesh,
        scratch_types=[
            pltpu.VMEM((CHUNK,), jnp.int32),
            pltpu.VMEM((CHUNK,), jnp.int32),
            pltpu.VMEM((CHUNK,), jnp.float32),
            pltpu.VMEM((CHUNK,), jnp.float32),
            pltpu.VMEM_SHARED((NP,), jnp.float32),
            pltpu.VMEM_SHARED((NP,), jnp.float32),
            pltpu.VMEM_SHARED((NP,), jnp.float32),
            pltpu.VMEM_SHARED((NP,), jnp.float32),
        ],
    )
    def edge_kernel(src_hbm, dst_hbm, u_hbm, dinv_hbm, zeros_hbm,
                    sin_out, sout_out,
                    isrc_v, idst_v, vals_v, vals2_v,
                    u_sh, dinv_sh, sin_sh, sout_sh):
        cid = lax.axis_index("c")
        sid = lax.axis_index("s")
        lo = sid * SL
        pltpu.sync_copy(u_hbm.at[pl.ds(lo, SL)], u_sh.at[pl.ds(lo, SL)])
        pltpu.sync_copy(dinv_hbm.at[pl.ds(lo, SL)],
                        dinv_sh.at[pl.ds(lo, SL)])
        pltpu.sync_copy(zeros_hbm.at[pl.ds(lo, SL)],
                        sin_sh.at[pl.ds(lo, SL)])
        pltpu.sync_copy(zeros_hbm.at[pl.ds(lo, SL)],
                        sout_sh.at[pl.ds(lo, SL)])
        plsc.subcore_barrier()
        base = (cid * NS + sid) * EPT

        def body(k, carry):
            off = base + k * CHUNK
            pltpu.sync_copy(src_hbm.at[pl.ds(off, CHUNK)], isrc_v)
            pltpu.sync_copy(dst_hbm.at[pl.ds(off, CHUNK)], idst_v)
            pltpu.sync_copy(u_sh.at[isrc_v], vals_v)
            pltpu.sync_copy(vals_v, sin_sh.at[idst_v], add=True)
            pltpu.sync_copy(dinv_sh.at[idst_v], vals2_v)
            pltpu.sync_copy(vals2_v, sout_sh.at[isrc_v], add=True)
            return carry

        lax.fori_loop(0, EPT // CHUNK, body, 0)
        plsc.subcore_barrier()
        pltpu.sync_copy(sin_sh.at[pl.ds(lo, SL)],
                        sin_out.at[cid, pl.ds(lo, SL)])
        pltpu.sync_copy(sout_sh.at[pl.ds(lo, SL)],
                        sout_out.at[cid, pl.ds(lo, SL)])

    return edge_kernel


def _norm_body(N, degp_ref, xp_ref, u_ref, dinv_ref):
    deg = degp_ref[0] + degp_ref[1] + 1.0
    R, L = deg.shape
    lin = (lax.broadcasted_iota(jnp.int32, (R, L), 0) * L
           + lax.broadcasted_iota(jnp.int32, (R, L), 1))
    dv = jnp.where(lin < N, lax.rsqrt(deg), 0.0)
    dinv_ref[...] = dv
    u_ref[...] = dv * xp_ref[...]


def _dense_body(N, nsteps, sinp_ref, soutp_ref, dinv_ref, xp_ref, w1_ref,
                b1_ref, w2_ref, b2_ref, out_ref, vacc_ref):
    i = pl.program_id(0)
    dv = dinv_ref[...]
    s_in = sinp_ref[0] + sinp_ref[1]
    s_out = soutp_ref[0] + soutp_ref[1]
    agg = dv * s_in + dv * dv * xp_ref[...]
    cc = dv * s_out + dv * dv
    m = jnp.maximum(agg * w1_ref[...] + b1_ref[...], 0.0)
    w = jnp.sum(m * cc, axis=0, keepdims=True)

    @pl.when(i == 0)
    def _():
        vacc_ref[...] = jnp.zeros_like(vacc_ref)

    vacc_ref[...] += w

    @pl.when(i == nsteps - 1)
    def _():
        out_ref[...] = (
            jnp.dot(vacc_ref[...] * (1.0 / N), w2_ref[...],
                    preferred_element_type=jnp.float32) + b2_ref[...])


def kernel(x, edge_index, W1, b1, W2, b2):
    N = x.shape[0]
    E = edge_index.shape[1]
    hid = W1.shape[1]
    dout = W2.shape[1]

    NP = _cdiv(N, 1024) * 1024
    R = NP // 128
    SL = NP // NS
    EPT = _cdiv(E, NT * CHUNK) * CHUNK
    EP = EPT * NT
    TB = 1024
    nsteps = NP // TB

    f32 = jnp.float32
    xf = jnp.concatenate([x[:, 0].astype(f32), jnp.zeros((NP - N,), f32)])
    pad_e = EP - E
    pad_rows = max(NP - N, 1)
    pad_idx = (N + (jnp.arange(pad_e, dtype=jnp.int32) % pad_rows)
               ).astype(jnp.int32)
    srcp = jnp.concatenate([edge_index[0].astype(jnp.int32), pad_idx])
    dstp = jnp.concatenate([edge_index[1].astype(jnp.int32), pad_idx])
    ones_c = jnp.ones((CHUNK,), f32)
    zeros_np = jnp.zeros((NP,), f32)

    degp = _make_deg_kernel(EP, NP, SL, EPT)(dstp, ones_c, zeros_np)

    u2d, dinv2d = pl.pallas_call(
        functools.partial(_norm_body, N),
        out_shape=(jax.ShapeDtypeStruct((R, 128), f32),
                   jax.ShapeDtypeStruct((R, 128), f32)),
    )(degp.reshape(NC, R, 128), xf.reshape(R, 128))
    u = u2d.reshape(NP)
    dinv = dinv2d.reshape(NP)

    sinp, soutp = _make_edge_kernel(EP, NP, SL, EPT)(
        srcp, dstp, u, dinv, zeros_np)

    out2d = pl.pallas_call(
        functools.partial(_dense_body, N, nsteps),
        grid=(nsteps,),
        in_specs=[
            pl.BlockSpec((NC, TB, 1), lambda i: (0, i, 0)),
            pl.BlockSpec((NC, TB, 1), lambda i: (0, i, 0)),
            pl.BlockSpec((TB, 1), lambda i: (i, 0)),
            pl.BlockSpec((TB, 1), lambda i: (i, 0)),
            pl.BlockSpec((1, hid), lambda i: (0, 0)),
            pl.BlockSpec((1, hid), lambda i: (0, 0)),
            pl.BlockSpec((hid, dout), lambda i: (0, 0)),
            pl.BlockSpec((1, dout), lambda i: (0, 0)),
        ],
        out_specs=pl.BlockSpec((1, dout), lambda i: (0, 0)),
        out_shape=jax.ShapeDtypeStruct((1, dout), f32),
        scratch_shapes=[pltpu.VMEM((1, hid), f32)],
    )(sinp.reshape(NC, NP, 1), soutp.reshape(NC, NP, 1),
      dinv.reshape(NP, 1), xf.reshape(NP, 1),
      W1.astype(f32), b1.reshape(1, hid).astype(f32),
      W2.astype(f32), b2.reshape(1, dout).astype(f32))

    return out2d.reshape(dout)


# speedup vs baseline: 114.2363x; 1.1280x over previous
import functools

import jax
import jax.numpy as jnp
from jax import lax
from jax.experimental import pallas as pl
from jax.experimental.pallas import tpu as pltpu
from jax.experimental.pallas import tpu_sc as plsc

NC = 2
NS = 16
NT = NC * NS
CHUNK = 5120


def _cdiv(a, b):
    return (a + b - 1) // b


def _rsqrt16(d):
    i = lax.bitcast_convert_type(d, jnp.int32)
    i = 0x5F3759DF - lax.shift_right_logical(i, 1)
    y = lax.bitcast_convert_type(i, jnp.float32)
    for _ in range(3):
        y = y * (1.5 - 0.5 * d * y * y)
    return y


def _make_sc_kernel(N, NP, SL, EPT, EPTA):
    mesh = plsc.VectorSubcoreMesh(
        core_axis_name="c", subcore_axis_name="s", num_cores=NC,
        num_subcores=NS)
    f32 = jnp.float32

    @functools.partial(
        pl.kernel,
        out_type=(jax.ShapeDtypeStruct((NC, NP), f32),
                  jax.ShapeDtypeStruct((NC, NP), f32),
                  jax.ShapeDtypeStruct((NP,), f32)),
        mesh=mesh,
        scratch_types=[
            pltpu.VMEM((CHUNK,), jnp.int32),
            pltpu.VMEM((CHUNK,), jnp.int32),
            pltpu.VMEM((CHUNK,), jnp.int32),
            pltpu.VMEM((CHUNK,), jnp.int32),
            pltpu.VMEM((CHUNK,), f32),
            pltpu.VMEM((CHUNK,), f32),
            pltpu.VMEM((CHUNK,), f32),
            pltpu.VMEM((CHUNK,), f32),
            pltpu.VMEM((CHUNK,), f32),
            pltpu.VMEM((SL,), f32),
            pltpu.VMEM((SL,), f32),
            pltpu.VMEM((SL,), f32),
            pltpu.VMEM((SL,), f32),
            pltpu.VMEM_SHARED((NP,), f32),
            pltpu.VMEM_SHARED((NP,), f32),
            pltpu.VMEM_SHARED((NP,), f32),
            pltpu.VMEM_SHARED((NP,), f32),
            pltpu.VMEM_SHARED((NP,), f32),
        ] + [pltpu.SemaphoreType.DMA] * 10,
    )
    def sc_kernel(src_hbm, dst_hbm, x_hbm, zeros_hbm, ones_hbm,
                  sin_out, sout_out, dinv_out,
                  idst0_v, idst1_v, isrc0_v, isrc1_v,
                  uval0_v, uval1_v, dval0_v, dval1_v, ones_v,
                  deg_v, x_v, dinv_v, u_v,
                  deg_sh, u_sh, dinv_sh, sin_sh, sout_sh,
                  sem_i0, sem_i1, sem_gu0, sem_gu1, sem_gd0, sem_gd1,
                  sem_s10, sem_s11, sem_s20, sem_s21):
        cid = lax.axis_index("c")
        sid = lax.axis_index("s")
        idst_v = (idst0_v, idst1_v)
        isrc_v = (isrc0_v, isrc1_v)
        uval_v = (uval0_v, uval1_v)
        dval_v = (dval0_v, dval1_v)
        sem_i = (sem_i0, sem_i1)
        sem_gu = (sem_gu0, sem_gu1)
        sem_gd = (sem_gd0, sem_gd1)
        sem_s1 = (sem_s10, sem_s11)
        sem_s2 = (sem_s20, sem_s21)

        if True:
            lo = sid * SL
            sl = lambda: pl.ds(lo, SL)
            pltpu.sync_copy(ones_hbm, ones_v)
            pltpu.sync_copy(zeros_hbm.at[sl()], deg_sh.at[sl()])
            pltpu.sync_copy(zeros_hbm.at[sl()], sin_sh.at[sl()])
            pltpu.sync_copy(zeros_hbm.at[sl()], sout_sh.at[sl()])
            plsc.subcore_barrier()

            nA = EPTA // CHUNK
            baseA = sid * EPTA
            idx_d = pltpu.async_copy(
                dst_hbm.at[pl.ds(baseA, CHUNK)], idst_v[0], sem_i[0])
            scat = [None, None]
            for k in range(nA):
                p = k & 1
                q = p ^ 1
                if k + 1 < nA:
                    if scat[q] is not None:
                        scat[q].wait()
                        scat[q] = None
                    idx_next = pltpu.async_copy(
                        dst_hbm.at[pl.ds(baseA + (k + 1) * CHUNK, CHUNK)],
                        idst_v[q], sem_i[q])
                idx_d.wait()
                scat[p] = pltpu.async_copy(
                    ones_v, deg_sh.at[idst_v[p]], sem_s1[p], add=True)
                if k + 1 < nA:
                    idx_d = idx_next
            for d in scat:
                if d is not None:
                    d.wait()
            plsc.subcore_barrier()

            pltpu.sync_copy(deg_sh.at[sl()], deg_v)
            pltpu.sync_copy(x_hbm.at[sl()], x_v)
            lanes = lax.iota(jnp.int32, 16)

            def a2_body(i, carry):
                off = pl.ds(i * 16, 16)
                d = deg_v[off] + 1.0
                y = _rsqrt16(d)
                gidx = lo + i * 16 + lanes
                y = jnp.where(gidx < N, y, 0.0)
                dinv_v[off] = y
                u_v[off] = y * x_v[off]
                return carry

            lax.fori_loop(0, SL // 16, a2_body, 0)
            pltpu.sync_copy(dinv_v, dinv_sh.at[sl()])
            pltpu.sync_copy(u_v, u_sh.at[sl()])

            @pl.when(cid == 0)
            def _():
                pltpu.sync_copy(dinv_v, dinv_out.at[sl()])

            plsc.subcore_barrier()

            nB = EPT // CHUNK
            baseB = (cid * NS + sid) * EPT
            i_s = pltpu.async_copy(
                src_hbm.at[pl.ds(baseB, CHUNK)], isrc_v[0], sem_i[0])
            i_d = pltpu.async_copy(
                dst_hbm.at[pl.ds(baseB, CHUNK)], idst_v[0], sem_i[0])
            s1 = [None, None]
            s2 = [None, None]
            for k in range(nB):
                p = k & 1
                q = p ^ 1
                if k + 1 < nB:
                    for s in (s1, s2):
                        if s[q] is not None:
                            s[q].wait()
                            s[q] = None
                    off = baseB + (k + 1) * CHUNK
                    i_sn = pltpu.async_copy(
                        src_hbm.at[pl.ds(off, CHUNK)], isrc_v[q],
                        sem_i[q])
                    i_dn = pltpu.async_copy(
                        dst_hbm.at[pl.ds(off, CHUNK)], idst_v[q],
                        sem_i[q])
                i_s.wait()
                i_d.wait()
                g_u = pltpu.async_copy(
                    u_sh.at[isrc_v[p]], uval_v[p], sem_gu[p])
                g_d = pltpu.async_copy(
                    dinv_sh.at[idst_v[p]], dval_v[p], sem_gd[p])
                g_u.wait()
                s1[p] = pltpu.async_copy(
                    uval_v[p], sin_sh.at[idst_v[p]], sem_s1[p],
                    add=True)
                g_d.wait()
                s2[p] = pltpu.async_copy(
                    dval_v[p], sout_sh.at[isrc_v[p]], sem_s2[p],
                    add=True)
                if k + 1 < nB:
                    i_s = i_sn
                    i_d = i_dn
            for s in (s1, s2):
                for d in s:
                    if d is not None:
                        d.wait()
            plsc.subcore_barrier()

            pltpu.sync_copy(sin_sh.at[sl()], sin_out.at[cid, sl()])
            pltpu.sync_copy(sout_sh.at[sl()], sout_out.at[cid, sl()])

    return sc_kernel


def _dense_body(N, nsteps, sinp_ref, soutp_ref, dinv_ref, xp_ref, w1_ref,
                b1_ref, w2_ref, b2_ref, out_ref, vacc_ref):
    i = pl.program_id(0)
    dv = dinv_ref[...]
    s_in = sinp_ref[0] + sinp_ref[1]
    s_out = soutp_ref[0] + soutp_ref[1]
    agg = dv * s_in + dv * dv * xp_ref[...]
    cc = dv * s_out + dv * dv
    m = jnp.maximum(agg * w1_ref[...] + b1_ref[...], 0.0)
    w = jnp.sum(m * cc, axis=0, keepdims=True)

    @pl.when(i == 0)
    def _():
        vacc_ref[...] = jnp.zeros_like(vacc_ref)

    vacc_ref[...] += w

    @pl.when(i == nsteps - 1)
    def _():
        out_ref[...] = (
            jnp.dot(vacc_ref[...] * (1.0 / N), w2_ref[...],
                    preferred_element_type=jnp.float32) + b2_ref[...])


def kernel(x, edge_index, W1, b1, W2, b2):
    N = x.shape[0]
    E = edge_index.shape[1]
    hid = W1.shape[1]
    dout = W2.shape[1]

    NP = _cdiv(N, 7168) * 7168
    SL = NP // NS
    EPT = _cdiv(E, NT * CHUNK) * CHUNK
    EP = EPT * NT
    EPTA = EP // NS
    NB = 7168
    nsteps = NP // NB

    f32 = jnp.float32
    xf = jnp.concatenate([x[:, 0].astype(f32), jnp.zeros((NP - N,), f32)])
    pad_e = EP - E
    pad_rows = max(NP - N, 1)
    pad_idx = (N + (jnp.arange(pad_e, dtype=jnp.int32) % pad_rows)
               ).astype(jnp.int32)
    srcp = jnp.concatenate([edge_index[0].astype(jnp.int32), pad_idx])
    dstp = jnp.concatenate([edge_index[1].astype(jnp.int32), pad_idx])
    ones_c = jnp.ones((CHUNK,), f32)
    zeros_np = jnp.zeros((NP,), f32)

    sinp, soutp, dinv = _make_sc_kernel(N, NP, SL, EPT, EPTA)(
        srcp, dstp, xf, zeros_np, ones_c)

    out2d = pl.pallas_call(
        functools.partial(_dense_body, N, nsteps),
        grid=(nsteps,),
        in_specs=[
            pl.BlockSpec((NC, NB, 1), lambda i: (0, i, 0)),
            pl.BlockSpec((NC, NB, 1), lambda i: (0, i, 0)),
            pl.BlockSpec((NB, 1), lambda i: (i, 0)),
            pl.BlockSpec((NB, 1), lambda i: (i, 0)),
            pl.BlockSpec((1, hid), lambda i: (0, 0)),
            pl.BlockSpec((1, hid), lambda i: (0, 0)),
            pl.BlockSpec((hid, dout), lambda i: (0, 0)),
            pl.BlockSpec((1, dout), lambda i: (0, 0)),
        ],
        out_specs=pl.BlockSpec((1, dout), lambda i: (0, 0)),
        out_shape=jax.ShapeDtypeStruct((1, dout), f32),
        scratch_shapes=[pltpu.VMEM((1, hid), f32)],
    )(sinp.reshape(NC, NP, 1), soutp.reshape(NC, NP, 1),
      dinv.reshape(NP, 1), xf.reshape(NP, 1),
      W1.astype(f32), b1.reshape(1, hid).astype(f32),
      W2.astype(f32), b2.reshape(1, dout).astype(f32))

    return out2d.reshape(dout)


# speedup vs baseline: 273.0111x; 2.3899x over previous
import functools

import jax
import jax.numpy as jnp
from jax import lax
from jax.experimental import pallas as pl
from jax.experimental.pallas import tpu as pltpu
from jax.experimental.pallas import tpu_sc as plsc

NC = 2
NS = 16
NT = NC * NS
CHUNK = 10000


def _cdiv(a, b):
    return (a + b - 1) // b


def _rsqrt16(d):
    i = lax.bitcast_convert_type(d, jnp.int32)
    i = 0x5F3759DF - lax.shift_right_logical(i, 1)
    y = lax.bitcast_convert_type(i, jnp.float32)
    for _ in range(3):
        y = y * (1.5 - 0.5 * d * y * y)
    return y


def _make_sc_kernel(N, NP, SL, EPT, EPTA, EOFF):
    mesh = plsc.VectorSubcoreMesh(
        core_axis_name="c", subcore_axis_name="s", num_cores=NC,
        num_subcores=NS)
    f32 = jnp.float32

    @functools.partial(
        pl.kernel,
        out_type=(jax.ShapeDtypeStruct((NC, NP), f32),
                  jax.ShapeDtypeStruct((NC, NP), f32),
                  jax.ShapeDtypeStruct((NP,), f32)),
        mesh=mesh,
        scratch_types=[
            pltpu.VMEM((CHUNK,), jnp.int32),
            pltpu.VMEM((CHUNK,), jnp.int32),
            pltpu.VMEM((CHUNK,), jnp.int32),
            pltpu.VMEM((CHUNK,), jnp.int32),
            pltpu.VMEM((CHUNK,), f32),
            pltpu.VMEM((CHUNK,), f32),
            pltpu.VMEM((CHUNK,), f32),
            pltpu.VMEM((CHUNK,), f32),
            pltpu.VMEM((SL,), f32),
            pltpu.VMEM((SL,), f32),
            pltpu.VMEM_SHARED((NP,), f32),
            pltpu.VMEM_SHARED((NP,), f32),
            pltpu.VMEM_SHARED((NP,), f32),
            pltpu.VMEM_SHARED((NP,), f32),
        ] + [pltpu.SemaphoreType.DMA] * 10,
    )
    def sc_kernel(ei_hbm, x_hbm, ones_hbm,
                  sin_out, sout_out, dinv_out,
                  idst0_v, idst1_v, isrc0_v, isrc1_v,
                  uval0_v, uval1_v, dval0_v, dval1_v,
                  deg_v, x_v,
                  u_sh, dinv_sh, sin_sh, sout_sh,
                  sem_i0, sem_i1, sem_gu0, sem_gu1, sem_gd0, sem_gd1,
                  sem_s10, sem_s11, sem_s20, sem_s21):
        cid = lax.axis_index("c")
        sid = lax.axis_index("s")
        idst_v = (idst0_v, idst1_v)
        isrc_v = (isrc0_v, isrc1_v)
        uval_v = (uval0_v, uval1_v)
        dval_v = (dval0_v, dval1_v)
        sem_i = (sem_i0, sem_i1)
        sem_gu = (sem_gu0, sem_gu1)
        sem_gd = (sem_gd0, sem_gd1)
        sem_s1 = (sem_s10, sem_s11)
        sem_s2 = (sem_s20, sem_s21)

        lo = sid * SL
        sl = lambda: pl.ds(lo, SL)
        ones_ld = pltpu.async_copy(ones_hbm, uval0_v, sem_gu0)

        zvec = jnp.zeros((16,), f32)

        def zfill(i, carry):
            deg_v[pl.ds(i * 16, 16)] = zvec
            return carry

        lax.fori_loop(0, SL // 16, zfill, 0)
        pltpu.sync_copy(deg_v, sin_sh.at[sl()])
        pltpu.sync_copy(deg_v, sout_sh.at[sl()])
        ones_ld.wait()
        plsc.subcore_barrier()

        nA = EPTA // CHUNK
        baseA = sid * EPTA
        idx_d = pltpu.async_copy(
            ei_hbm.at[pl.ds(EOFF + baseA, CHUNK)], idst_v[0], sem_i[0])
        scat = [None, None]
        for k in range(nA):
            p = k & 1
            q = p ^ 1
            if k + 1 < nA:
                if scat[q] is not None:
                    scat[q].wait()
                    scat[q] = None
                idx_next = pltpu.async_copy(
                    ei_hbm.at[pl.ds(EOFF + baseA + (k + 1) * CHUNK, CHUNK)],
                    idst_v[q], sem_i[q])
            idx_d.wait()
            scat[p] = pltpu.async_copy(
                uval0_v, sout_sh.at[idst_v[p]], sem_s1[p], add=True)
            if k + 1 < nA:
                idx_d = idx_next
        for d in scat:
            if d is not None:
                d.wait()
        plsc.subcore_barrier()

        pltpu.sync_copy(sout_sh.at[sl()], deg_v)
        pltpu.sync_copy(x_hbm.at[sl()], x_v)
        lanes = lax.iota(jnp.int32, 16)

        def a2_body(i, carry):
            off = pl.ds(i * 16, 16)
            d = deg_v[off] + 1.0
            y = _rsqrt16(d)
            gidx = lo + i * 16 + lanes
            y = jnp.where(gidx < N, y, 0.0)
            deg_v[off] = y
            x_v[off] = y * x_v[off]
            return carry

        lax.fori_loop(0, SL // 16, a2_body, 0)
        pltpu.sync_copy(deg_v, dinv_sh.at[sl()])
        pltpu.sync_copy(x_v, u_sh.at[sl()])

        @pl.when(cid == 0)
        def _():
            pltpu.sync_copy(deg_v, dinv_out.at[sl()])

        def zfill2(i, carry):
            x_v[pl.ds(i * 16, 16)] = zvec
            return carry

        lax.fori_loop(0, SL // 16, zfill2, 0)
        pltpu.sync_copy(x_v, sout_sh.at[sl()])
        plsc.subcore_barrier()

        nB = EPT // CHUNK
        baseB = (cid * NS + sid) * EPT
        i_s = pltpu.async_copy(
            ei_hbm.at[pl.ds(baseB, CHUNK)], isrc_v[0], sem_i[0])
        i_d = pltpu.async_copy(
            ei_hbm.at[pl.ds(EOFF + baseB, CHUNK)], idst_v[0], sem_i[0])
        s1 = [None, None]
        s2 = [None, None]
        for k in range(nB):
            p = k & 1
            q = p ^ 1
            if k + 1 < nB:
                for s in (s1, s2):
                    if s[q] is not None:
                        s[q].wait()
                        s[q] = None
                off = baseB + (k + 1) * CHUNK
                i_sn = pltpu.async_copy(
                    ei_hbm.at[pl.ds(off, CHUNK)], isrc_v[q], sem_i[q])
                i_dn = pltpu.async_copy(
                    ei_hbm.at[pl.ds(EOFF + off, CHUNK)], idst_v[q], sem_i[q])
            i_s.wait()
            i_d.wait()
            g_u = pltpu.async_copy(
                u_sh.at[isrc_v[p]], uval_v[p], sem_gu[p])
            g_d = pltpu.async_copy(
                dinv_sh.at[idst_v[p]], dval_v[p], sem_gd[p])
            g_u.wait()
            s1[p] = pltpu.async_copy(
                uval_v[p], sin_sh.at[idst_v[p]], sem_s1[p], add=True)
            g_d.wait()
            s2[p] = pltpu.async_copy(
                dval_v[p], sout_sh.at[isrc_v[p]], sem_s2[p], add=True)
            if k + 1 < nB:
                i_s = i_sn
                i_d = i_dn
        for s in (s1, s2):
            for d in s:
                if d is not None:
                    d.wait()
        plsc.subcore_barrier()

        pltpu.sync_copy(sin_sh.at[sl()], sin_out.at[cid, sl()])
        pltpu.sync_copy(sout_sh.at[sl()], sout_out.at[cid, sl()])

    return sc_kernel


def _dense_body(N, nsteps, sinp_ref, soutp_ref, dinv_ref, xp_ref, w1_ref,
                b1_ref, w2_ref, b2_ref, out_ref, vacc_ref):
    i = pl.program_id(0)
    dv = dinv_ref[0]
    s_in = sinp_ref[0, 0] + sinp_ref[1, 0]
    s_out = soutp_ref[0, 0] + soutp_ref[1, 0]
    agg = dv * s_in + dv * dv * xp_ref[0]
    cc = dv * s_out + dv * dv
    m = jnp.maximum(w1_ref[...] * agg + b1_ref[...], 0.0)
    w = jnp.sum(m * cc, axis=1, keepdims=True)

    @pl.when(i == 0)
    def _():
        vacc_ref[...] = jnp.zeros_like(vacc_ref)

    vacc_ref[...] += w

    @pl.when(i == nsteps - 1)
    def _():
        out_ref[...] = (
            lax.dot_general(vacc_ref[...] * (1.0 / N), w2_ref[...],
                            (((0,), (0,)), ((), ())),
                            preferred_element_type=jnp.float32)
            + b2_ref[...])


def kernel(x, edge_index, W1, b1, W2, b2):
    N = x.shape[0]
    E = edge_index.shape[1]
    hid = W1.shape[1]
    dout = W2.shape[1]

    NB = 7168
    NP = _cdiv(N, NB) * NB
    SL = NP // NS
    nsteps = NP // NB
    assert E % (NT * CHUNK) == 0, (E, NT * CHUNK)
    EPT = E // NT
    EPTA = E // NS

    f32 = jnp.float32
    xf = jnp.concatenate([x[:, 0].astype(f32), jnp.zeros((NP - N,), f32)])
    ei = edge_index.astype(jnp.int32)
    ones_c = jnp.ones((CHUNK,), f32)

    sinp, soutp, dinv = _make_sc_kernel(N, NP, SL, EPT, EPTA, E)(
        ei.reshape(2 * E), xf, ones_c)

    out2d = pl.pallas_call(
        functools.partial(_dense_body, N, nsteps),
        grid=(nsteps,),
        in_specs=[
            pl.BlockSpec((NC, 1, 1, NB), lambda i: (0, i, 0, 0)),
            pl.BlockSpec((NC, 1, 1, NB), lambda i: (0, i, 0, 0)),
            pl.BlockSpec((1, 1, NB), lambda i: (i, 0, 0)),
            pl.BlockSpec((1, 1, NB), lambda i: (i, 0, 0)),
            pl.BlockSpec((hid, 1), lambda i: (0, 0)),
            pl.BlockSpec((hid, 1), lambda i: (0, 0)),
            pl.BlockSpec((hid, dout), lambda i: (0, 0)),
            pl.BlockSpec((1, dout), lambda i: (0, 0)),
        ],
        out_specs=pl.BlockSpec((1, dout), lambda i: (0, 0)),
        out_shape=jax.ShapeDtypeStruct((1, dout), f32),
        scratch_shapes=[pltpu.VMEM((hid, 1), f32)],
    )(sinp.reshape(NC, nsteps, 1, NB), soutp.reshape(NC, nsteps, 1, NB),
      dinv.reshape(nsteps, 1, NB), xf.reshape(nsteps, 1, NB),
      W1.reshape(hid, 1).astype(f32), b1.reshape(hid, 1).astype(f32),
      W2.astype(f32), b2.reshape(1, dout).astype(f32))

    return out2d.reshape(dout)


# speedup vs baseline: 286.1685x; 1.0482x over previous
import functools

import jax
import jax.numpy as jnp
from jax import lax
from jax.experimental import pallas as pl
from jax.experimental.pallas import tpu as pltpu
from jax.experimental.pallas import tpu_sc as plsc

NC = 2
NS = 16
NT = NC * NS
CHUNK = 10000


def _cdiv(a, b):
    return (a + b - 1) // b


def _rsqrt16(d):
    i = lax.bitcast_convert_type(d, jnp.int32)
    i = 0x5F3759DF - lax.shift_right_logical(i, 1)
    y = lax.bitcast_convert_type(i, jnp.float32)
    for _ in range(2):
        y = y * (1.5 - 0.5 * d * y * y)
    return y


def _make_sc_kernel(N, NP, SL, EPT, EPTA, EOFF):
    mesh = plsc.VectorSubcoreMesh(
        core_axis_name="c", subcore_axis_name="s", num_cores=NC,
        num_subcores=NS)
    f32 = jnp.float32

    @functools.partial(
        pl.kernel,
        out_type=(jax.ShapeDtypeStruct((NC, NP), f32),
                  jax.ShapeDtypeStruct((NC, NP), f32),
                  jax.ShapeDtypeStruct((1, NP), f32)),
        mesh=mesh,
        scratch_types=[
            pltpu.VMEM((CHUNK,), jnp.int32),
            pltpu.VMEM((CHUNK,), jnp.int32),
            pltpu.VMEM((CHUNK,), jnp.int32),
            pltpu.VMEM((CHUNK,), jnp.int32),
            pltpu.VMEM((CHUNK,), f32),
            pltpu.VMEM((CHUNK,), f32),
            pltpu.VMEM((CHUNK,), f32),
            pltpu.VMEM((CHUNK,), f32),
            pltpu.VMEM((SL,), f32),
            pltpu.VMEM((SL,), f32),
            pltpu.VMEM_SHARED((NP,), f32),
            pltpu.VMEM_SHARED((NP,), f32),
            pltpu.VMEM_SHARED((NP,), f32),
            pltpu.VMEM_SHARED((NP,), f32),
        ] + [pltpu.SemaphoreType.DMA] * 10,
    )
    def sc_kernel(ei_hbm, x_hbm, ones_hbm,
                  sin_out, sout_out, dinv_out,
                  idst0_v, idst1_v, isrc0_v, isrc1_v,
                  uval0_v, uval1_v, dval0_v, dval1_v,
                  deg_v, x_v,
                  u_sh, dinv_sh, sin_sh, sout_sh,
                  sem_i0, sem_i1, sem_gu0, sem_gu1, sem_gd0, sem_gd1,
                  sem_s10, sem_s11, sem_s20, sem_s21):
        cid = lax.axis_index("c")
        sid = lax.axis_index("s")
        idst_v = (idst0_v, idst1_v)
        isrc_v = (isrc0_v, isrc1_v)
        uval_v = (uval0_v, uval1_v)
        dval_v = (dval0_v, dval1_v)
        sem_i = (sem_i0, sem_i1)
        sem_gu = (sem_gu0, sem_gu1)
        sem_gd = (sem_gd0, sem_gd1)
        sem_s1 = (sem_s10, sem_s11)
        sem_s2 = (sem_s20, sem_s21)

        lo = sid * SL
        sl = lambda: pl.ds(lo, SL)
        ones_ld = pltpu.async_copy(ones_hbm, uval0_v, sem_gu0)

        zvec = jnp.zeros((16,), f32)

        def zfill(i, carry):
            deg_v[pl.ds(i * 16, 16)] = zvec
            return carry

        lax.fori_loop(0, SL // 16, zfill, 0)
        pltpu.sync_copy(deg_v, sin_sh.at[sl()])
        pltpu.sync_copy(deg_v, sout_sh.at[sl()])
        ones_ld.wait()
        plsc.subcore_barrier()

        nA = EPTA // CHUNK
        baseA = sid * EPTA
        idx_d = pltpu.async_copy(
            ei_hbm.at[pl.ds(EOFF + baseA, CHUNK)], idst_v[0], sem_i[0])
        scat = [None, None]
        for k in range(nA):
            p = k & 1
            q = p ^ 1
            if k + 1 < nA:
                if scat[q] is not None:
                    scat[q].wait()
                    scat[q] = None
                idx_next = pltpu.async_copy(
                    ei_hbm.at[pl.ds(EOFF + baseA + (k + 1) * CHUNK, CHUNK)],
                    idst_v[q], sem_i[q])
            idx_d.wait()
            scat[p] = pltpu.async_copy(
                uval0_v, sout_sh.at[idst_v[p]], sem_s1[p], add=True)
            if k + 1 < nA:
                idx_d = idx_next
        for d in scat:
            if d is not None:
                d.wait()
        plsc.subcore_barrier()

        pltpu.sync_copy(sout_sh.at[sl()], deg_v)
        pltpu.sync_copy(x_hbm.at[0, sl()], x_v)
        lanes = lax.iota(jnp.int32, 16)

        def a2_body(i, carry):
            off = pl.ds(i * 16, 16)
            d = deg_v[off] + 1.0
            y = _rsqrt16(d)
            gidx = lo + i * 16 + lanes
            y = jnp.where(gidx < N, y, 0.0)
            deg_v[off] = y
            x_v[off] = y * x_v[off]
            return carry

        lax.fori_loop(0, SL // 16, a2_body, 0)
        pltpu.sync_copy(deg_v, dinv_sh.at[sl()])
        pltpu.sync_copy(x_v, u_sh.at[sl()])

        @pl.when(cid == 0)
        def _():
            pltpu.sync_copy(deg_v, dinv_out.at[0, sl()])

        def zfill2(i, carry):
            x_v[pl.ds(i * 16, 16)] = zvec
            return carry

        lax.fori_loop(0, SL // 16, zfill2, 0)
        pltpu.sync_copy(x_v, sout_sh.at[sl()])
        plsc.subcore_barrier()

        nB = EPT // CHUNK
        baseB = (cid * NS + sid) * EPT
        i_s = pltpu.async_copy(
            ei_hbm.at[pl.ds(baseB, CHUNK)], isrc_v[0], sem_i[0])
        i_d = pltpu.async_copy(
            ei_hbm.at[pl.ds(EOFF + baseB, CHUNK)], idst_v[0], sem_i[0])
        s1 = [None, None]
        s2 = [None, None]
        for k in range(nB):
            p = k & 1
            q = p ^ 1
            if k + 1 < nB:
                for s in (s1, s2):
                    if s[q] is not None:
                        s[q].wait()
                        s[q] = None
                off = baseB + (k + 1) * CHUNK
                i_sn = pltpu.async_copy(
                    ei_hbm.at[pl.ds(off, CHUNK)], isrc_v[q], sem_i[q])
                i_dn = pltpu.async_copy(
                    ei_hbm.at[pl.ds(EOFF + off, CHUNK)], idst_v[q], sem_i[q])
            i_s.wait()
            i_d.wait()
            g_u = pltpu.async_copy(
                u_sh.at[isrc_v[p]], uval_v[p], sem_gu[p])
            g_d = pltpu.async_copy(
                dinv_sh.at[idst_v[p]], dval_v[p], sem_gd[p])
            g_u.wait()
            s1[p] = pltpu.async_copy(
                uval_v[p], sin_sh.at[idst_v[p]], sem_s1[p], add=True)
            g_d.wait()
            s2[p] = pltpu.async_copy(
                dval_v[p], sout_sh.at[isrc_v[p]], sem_s2[p], add=True)
            if k + 1 < nB:
                i_s = i_sn
                i_d = i_dn
        for s in (s1, s2):
            for d in s:
                if d is not None:
                    d.wait()
        plsc.subcore_barrier()

        pltpu.sync_copy(sin_sh.at[sl()], sin_out.at[cid, sl()])
        pltpu.sync_copy(sout_sh.at[sl()], sout_out.at[cid, sl()])

    return sc_kernel


def _dense_body(N, nsteps, sinp_ref, soutp_ref, dinv_ref, xp_ref, w1_ref,
                b1_ref, w2_ref, b2_ref, out_ref, vacc_ref):
    i = pl.program_id(0)
    dv = dinv_ref[...]
    s_in = sinp_ref[0:1, :] + sinp_ref[1:2, :]
    s_out = soutp_ref[0:1, :] + soutp_ref[1:2, :]
    agg = dv * s_in + dv * dv * xp_ref[...]
    cc = dv * s_out + dv * dv
    m = jnp.maximum(w1_ref[...] * agg + b1_ref[...], 0.0)
    w = jnp.sum(m * cc, axis=1, keepdims=True)

    @pl.when(i == 0)
    def _():
        vacc_ref[...] = jnp.zeros_like(vacc_ref)

    vacc_ref[...] += w

    @pl.when(i == nsteps - 1)
    def _():
        out_ref[...] = (
            lax.dot_general(vacc_ref[...] * (1.0 / N), w2_ref[...],
                            (((0,), (0,)), ((), ())),
                            preferred_element_type=jnp.float32)
            + b2_ref[...])


def kernel(x, edge_index, W1, b1, W2, b2):
    N = x.shape[0]
    E = edge_index.shape[1]
    hid = W1.shape[1]
    dout = W2.shape[1]

    NB = 7168
    NP = _cdiv(N, NB) * NB
    SL = NP // NS
    nsteps = NP // NB
    assert E % (NT * CHUNK) == 0, (E, NT * CHUNK)
    EPT = E // NT
    EPTA = E // NS

    f32 = jnp.float32
    xf = jnp.concatenate([x[:, 0].astype(f32),
                          jnp.zeros((NP - N,), f32)]).reshape(1, NP)
    ei = edge_index.astype(jnp.int32)
    ones_c = jnp.ones((CHUNK,), f32)

    sinp, soutp, dinv = _make_sc_kernel(N, NP, SL, EPT, EPTA, E)(
        ei.reshape(2 * E), xf, ones_c)

    out2d = pl.pallas_call(
        functools.partial(_dense_body, N, nsteps),
        grid=(nsteps,),
        in_specs=[
            pl.BlockSpec((NC, NB), lambda i: (0, i)),
            pl.BlockSpec((NC, NB), lambda i: (0, i)),
            pl.BlockSpec((1, NB), lambda i: (0, i)),
            pl.BlockSpec((1, NB), lambda i: (0, i)),
            pl.BlockSpec((hid, 1), lambda i: (0, 0)),
            pl.BlockSpec((hid, 1), lambda i: (0, 0)),
            pl.BlockSpec((hid, dout), lambda i: (0, 0)),
            pl.BlockSpec((1, dout), lambda i: (0, 0)),
        ],
        out_specs=pl.BlockSpec((1, dout), lambda i: (0, 0)),
        out_shape=jax.ShapeDtypeStruct((1, dout), f32),
        scratch_shapes=[pltpu.VMEM((hid, 1), f32)],
    )(sinp, soutp, dinv, xf,
      W1.reshape(hid, 1).astype(f32), b1.reshape(hid, 1).astype(f32),
      W2.astype(f32), b2.reshape(1, dout).astype(f32))

    return out2d.reshape(dout)
